# baseline (speedup 1.0000x reference)
import jax
import jax.numpy as jnp
from jax import lax
from jax.experimental import pallas as pl
from jax.experimental.pallas import tpu as pltpu
from jax.experimental.pallas import tpu_sc as plsc

_N_ATOMS = 100000
_EMB = 128
_N_ELEM = 94
_INFO = plsc.get_sparse_core_info()
_NW = _INFO.num_cores * _INFO.num_subcores
_CHUNK = 128
_SLAB = 3128
_SLAB_LAST = _N_ATOMS - (_NW - 1) * _SLAB
_IDX_PAD = 3136
_NBUF = 4
_LAG = 2

_SIZES_MAIN = [_CHUNK] * 24 + [56]
_SIZES_LAST = [_CHUNK] * 23 + [88]
_OFFS = [i * _CHUNK for i in range(25)]


def _pipeline(base, sizes, idx_v, w_v, out_hbm, bufs, gsem, ssem):
    nch = len(sizes)

    def gather(j):
        n = sizes[j]
        for b in range(-(-n // 16)):
            w = pl.ds(_OFFS[j] + 16 * b, 16)
            idx_v[w] = idx_v[w] - 1
        pltpu.async_copy(w_v.at[idx_v.at[pl.ds(_OFFS[j], n)]],
                         bufs[j % _NBUF].at[pl.ds(0, n)], gsem)

    def gather_wait(j):
        n = sizes[j]
        pltpu.make_async_copy(w_v.at[idx_v.at[pl.ds(_OFFS[j], n)]],
                              bufs[j % _NBUF].at[pl.ds(0, n)], gsem).wait()

    def scatter(j, wait):
        n = sizes[j]
        src = bufs[j % _NBUF].at[pl.ds(0, n)]
        dst = out_hbm.at[pl.ds(pl.multiple_of(base + _OFFS[j], 8), n)]
        if wait:
            pltpu.make_async_copy(src, dst, ssem).wait()
        else:
            pltpu.async_copy(src, dst, ssem)

    for j in range(_LAG):
        gather(j)
    for j in range(nch):
        gather_wait(j)
        scatter(j, wait=False)
        if j >= _LAG:
            scatter(j - _LAG, wait=True)
        if j + _LAG < nch:
            gather(j + _LAG)
    for j in range(nch - _LAG, nch):
        scatter(j, wait=True)


def _body(z_hbm, w_hbm, out_hbm, idx_v, w_v, b0, b1, b2, b3, gsem, ssem):
    bufs = (b0, b1, b2, b3)
    wid = lax.axis_index("s") * _INFO.num_cores + lax.axis_index("c")
    base = pl.multiple_of(wid * _SLAB, 8)

    @pl.when(lax.axis_index("s") == 0)
    def _():
        pltpu.sync_copy(w_hbm, w_v)

    plsc.subcore_barrier()

    @pl.when(wid < _NW - 1)
    def _():
        pltpu.sync_copy(z_hbm.at[pl.ds(base, _SLAB)],
                        idx_v.at[pl.ds(0, _SLAB)])
        _pipeline(base, _SIZES_MAIN, idx_v, w_v, out_hbm, bufs, gsem, ssem)

    @pl.when(wid == _NW - 1)
    def _():
        pltpu.sync_copy(z_hbm.at[pl.ds(base, _SLAB_LAST)],
                        idx_v.at[pl.ds(0, _SLAB_LAST)])
        _pipeline(base, _SIZES_LAST, idx_v, w_v, out_hbm, bufs, gsem, ssem)


@jax.jit
def kernel(Z, W):
    mesh = plsc.VectorSubcoreMesh(core_axis_name="c", subcore_axis_name="s")
    return pl.kernel(
        _body,
        out_type=jax.ShapeDtypeStruct((_N_ATOMS, _EMB), jnp.float32),
        mesh=mesh,
        scratch_types=[
            pltpu.VMEM((_IDX_PAD,), jnp.int32),
            pltpu.VMEM_SHARED((_N_ELEM, _EMB), jnp.float32),
            pltpu.VMEM((_CHUNK, _EMB), jnp.float32),
            pltpu.VMEM((_CHUNK, _EMB), jnp.float32),
            pltpu.VMEM((_CHUNK, _EMB), jnp.float32),
            pltpu.VMEM((_CHUNK, _EMB), jnp.float32),
            pltpu.SemaphoreType.DMA,
            pltpu.SemaphoreType.DMA,
        ],
    )(Z, W)


# speedup vs baseline: 1.0081x; 1.0081x over previous
import jax
import jax.numpy as jnp
from jax import lax
from jax.experimental import pallas as pl
from jax.experimental.pallas import tpu as pltpu
from jax.experimental.pallas import tpu_sc as plsc

_N_ATOMS = 100000
_EMB = 128
_N_ELEM = 94
_INFO = plsc.get_sparse_core_info()
_NW = _INFO.num_cores * _INFO.num_subcores
_GMAX = 128
_CHUNK = 256
_SLAB = 3128
_SLAB_LAST = _N_ATOMS - (_NW - 1) * _SLAB
_NBUF = 3
_LAG = 1

_SIZES_MAIN = [_CHUNK] * 12 + [56]
_SIZES_LAST = [_CHUNK] * 11 + [216]
_OFFS = [i * _CHUNK for i in range(13)]


def _pipeline(base, sizes, idx_v, w_v, out_hbm, bufs, gsem, ssem):
    nch = len(sizes)

    def gather_parts(j):
        n = sizes[j]
        parts = []
        p = 0
        while p < n:
            m = min(_GMAX, n - p)
            parts.append((p, m))
            p += m
        return parts

    def gather(j):
        for p, m in gather_parts(j):
            pltpu.async_copy(w_v.at[idx_v.at[pl.ds(_OFFS[j] + p, m)]],
                             bufs[j % _NBUF].at[pl.ds(p, m)], gsem)

    def gather_wait(j):
        for p, m in gather_parts(j):
            pltpu.make_async_copy(w_v.at[idx_v.at[pl.ds(_OFFS[j] + p, m)]],
                                  bufs[j % _NBUF].at[pl.ds(p, m)],
                                  gsem).wait()

    def scatter(j, wait):
        n = sizes[j]
        src = bufs[j % _NBUF].at[pl.ds(0, n)]
        dst = out_hbm.at[pl.ds(pl.multiple_of(base + _OFFS[j], 8), n)]
        if wait:
            pltpu.make_async_copy(src, dst, ssem).wait()
        else:
            pltpu.async_copy(src, dst, ssem)

    for j in range(_LAG):
        gather(j)
    for j in range(nch):
        gather_wait(j)
        scatter(j, wait=False)
        if j >= _LAG:
            scatter(j - _LAG, wait=True)
        if j + _LAG < nch:
            gather(j + _LAG)
    for j in range(nch - _LAG, nch):
        scatter(j, wait=True)


def _body(z_hbm, w_hbm, out_hbm, idx_v, w_v, b0, b1, b2, gsem, ssem):
    bufs = (b0, b1, b2)
    wid = lax.axis_index("s") * _INFO.num_cores + lax.axis_index("c")
    base = pl.multiple_of(wid * _SLAB, 8)

    @pl.when(lax.axis_index("s") == 0)
    def _():
        pltpu.sync_copy(w_hbm, w_v)

    plsc.subcore_barrier()

    @pl.when(wid < _NW - 1)
    def _():
        pltpu.sync_copy(z_hbm.at[pl.ds(base, _SLAB)], idx_v)
        _pipeline(base, _SIZES_MAIN, idx_v, w_v, out_hbm, bufs, gsem, ssem)

    @pl.when(wid == _NW - 1)
    def _():
        pltpu.sync_copy(z_hbm.at[pl.ds(base, _SLAB_LAST)],
                        idx_v.at[pl.ds(0, _SLAB_LAST)])
        _pipeline(base, _SIZES_LAST, idx_v, w_v, out_hbm, bufs, gsem, ssem)


@jax.jit
def kernel(Z, W):
    w95 = jnp.concatenate([jnp.zeros((1, _EMB), jnp.float32), W])
    mesh = plsc.VectorSubcoreMesh(core_axis_name="c", subcore_axis_name="s")
    return pl.kernel(
        _body,
        out_type=jax.ShapeDtypeStruct((_N_ATOMS, _EMB), jnp.float32),
        mesh=mesh,
        scratch_types=[
            pltpu.VMEM((_SLAB,), jnp.int32),
            pltpu.VMEM_SHARED((_N_ELEM + 1, _EMB), jnp.float32),
            pltpu.VMEM((_CHUNK, _EMB), jnp.float32),
            pltpu.VMEM((_CHUNK, _EMB), jnp.float32),
            pltpu.VMEM((_CHUNK, _EMB), jnp.float32),
            pltpu.SemaphoreType.DMA,
            pltpu.SemaphoreType.DMA,
        ],
    )(Z, w95)


# speedup vs baseline: 1.0368x; 1.0285x over previous
import jax
import jax.numpy as jnp
from jax import lax
from jax.experimental import pallas as pl
from jax.experimental.pallas import tpu as pltpu
from jax.experimental.pallas import tpu_sc as plsc

_N_ATOMS = 100000
_EMB = 128
_N_ELEM = 94
_INFO = plsc.get_sparse_core_info()
_NW = _INFO.num_cores * _INFO.num_subcores
_CHUNK = 128
_SLAB = 3128
_SLAB_LAST = _N_ATOMS - (_NW - 1) * _SLAB
_NBUF = 4
_LAG = 2

_SIZES_MAIN = [_CHUNK] * 24 + [56]
_SIZES_LAST = [_CHUNK] * 23 + [88]
_OFFS = [i * _CHUNK for i in range(25)]


def _pipeline(base, sizes, idx_v, w_v, out_hbm, bufs, gsem, ssem):
    nch = len(sizes)

    def gather(j):
        n = sizes[j]
        pltpu.async_copy(w_v.at[idx_v.at[pl.ds(_OFFS[j], n)]],
                         bufs[j % _NBUF].at[pl.ds(0, n)], gsem)

    def gather_wait(j):
        n = sizes[j]
        pltpu.make_async_copy(w_v.at[idx_v.at[pl.ds(_OFFS[j], n)]],
                              bufs[j % _NBUF].at[pl.ds(0, n)], gsem).wait()

    def scatter(j, wait):
        n = sizes[j]
        src = bufs[j % _NBUF].at[pl.ds(0, n)]
        dst = out_hbm.at[pl.ds(pl.multiple_of(base + _OFFS[j], 8), n)]
        if wait:
            pltpu.make_async_copy(src, dst, ssem).wait()
        else:
            pltpu.async_copy(src, dst, ssem)

    for j in range(_LAG):
        gather(j)
    for j in range(nch):
        gather_wait(j)
        scatter(j, wait=False)
        if j >= _LAG:
            scatter(j - _LAG, wait=True)
        if j + _LAG < nch:
            gather(j + _LAG)
    for j in range(nch - _LAG, nch):
        scatter(j, wait=True)


def _body(z_hbm, w_hbm, out_hbm, idx_v, w_v, b0, b1, b2, b3, gsem, ssem):
    bufs = (b0, b1, b2, b3)
    wid = lax.axis_index("s") * _INFO.num_cores + lax.axis_index("c")
    base = pl.multiple_of(wid * _SLAB, 8)

    @pl.when(lax.axis_index("s") == 0)
    def _():
        pltpu.sync_copy(w_hbm, w_v)

    plsc.subcore_barrier()

    @pl.when(wid < _NW - 1)
    def _():
        pltpu.sync_copy(z_hbm.at[pl.ds(base, _SLAB)], idx_v)
        _pipeline(base, _SIZES_MAIN, idx_v, w_v, out_hbm, bufs, gsem, ssem)

    @pl.when(wid == _NW - 1)
    def _():
        pltpu.sync_copy(z_hbm.at[pl.ds(base, _SLAB_LAST)],
                        idx_v.at[pl.ds(0, _SLAB_LAST)])
        _pipeline(base, _SIZES_LAST, idx_v, w_v, out_hbm, bufs, gsem, ssem)


@jax.jit
def kernel(Z, W):
    w95 = jnp.concatenate([jnp.zeros((1, _EMB), jnp.float32), W])
    mesh = plsc.VectorSubcoreMesh(core_axis_name="c", subcore_axis_name="s")
    return pl.kernel(
        _body,
        out_type=jax.ShapeDtypeStruct((_N_ATOMS, _EMB), jnp.float32),
        mesh=mesh,
        scratch_types=[
            pltpu.VMEM((_SLAB,), jnp.int32),
            pltpu.VMEM_SHARED((_N_ELEM + 1, _EMB), jnp.float32),
            pltpu.VMEM((_CHUNK, _EMB), jnp.float32),
            pltpu.VMEM((_CHUNK, _EMB), jnp.float32),
            pltpu.VMEM((_CHUNK, _EMB), jnp.float32),
            pltpu.VMEM((_CHUNK, _EMB), jnp.float32),
            pltpu.SemaphoreType.DMA,
            pltpu.SemaphoreType.DMA,
        ],
    )(Z, w95)
